# baseline (speedup 1.0000x reference)
import functools

import jax
import jax.numpy as jnp
from jax import lax
from jax.experimental import pallas as pl
from jax.experimental.pallas import tpu as pltpu
from jax.experimental.pallas import tpu_sc as plsc

NC = 2
NS = 16
NW = NC * NS
LANES = 16
IDXW = 128
FW = 128


def _dot_t(a, w):
    return lax.dot_general(a, w, (((1,), (1,)), ((), ())),
                           preferred_element_type=jnp.float32)


K = 8


def _seg_sum_sc(y, src2, dst2, n_pad, ept):
    n_steps = ept // IDXW
    n_groups = n_steps // K
    zr = n_pad // NS
    mesh = plsc.VectorSubcoreMesh(core_axis_name="c", subcore_axis_name="s")

    @functools.partial(
        pl.kernel,
        out_type=jax.ShapeDtypeStruct((NC, n_pad, FW), jnp.float32),
        mesh=mesh,
        scratch_types=[
            pltpu.VMEM((K, IDXW), jnp.int32),
            pltpu.VMEM((K, IDXW), jnp.int32),
            pltpu.VMEM((2, IDXW, FW), jnp.float32),
            pltpu.VMEM_SHARED((n_pad, FW), jnp.float32),
            pltpu.SemaphoreType.DMA,
            pltpu.SemaphoreType.DMA,
            pltpu.SemaphoreType.DMA,
            pltpu.SemaphoreType.DMA,
        ],
    )
    def k(y_hbm, src_hbm, dst_hbm, out_hbm, src8, dst8, rows, acc,
          gsem0, gsem1, ssem0, ssem1):
        c = lax.axis_index("c")
        s = lax.axis_index("s")
        wid = s * NC + c
        off = pl.multiple_of(s * zr, 8)
        gsem = (gsem0, gsem1)
        ssem = (ssem0, ssem1)

        def zrow(i, carry):
            for j in range(FW // LANES):
                rows[0, i, pl.ds(j * LANES, LANES)] = jnp.zeros((LANES,),
                                                               jnp.float32)
            return carry
        lax.fori_loop(0, IDXW, zrow, 0)
        for zo in range(0, zr, IDXW):
            zn = min(IDXW, zr - zo)
            pltpu.sync_copy(rows.at[0, pl.ds(0, zn)],
                            acc.at[pl.ds(pl.multiple_of(off + zo, 8), zn)])
        plsc.subcore_barrier()

        tile_r0 = wid * (ept // IDXW)

        def group(g, carry):
            @pl.when(g > 0)
            def _():
                for b in range(2):
                    pltpu.make_async_copy(y_hbm.at[pl.ds(0, IDXW)],
                                          rows.at[b], ssem[b]).wait()

            r0 = pl.multiple_of(tile_r0 + g * K, 8)
            pltpu.sync_copy(src_hbm.at[pl.ds(r0, K)], src8)
            pltpu.sync_copy(dst_hbm.at[pl.ds(r0, K)], dst8)

            gd = pltpu.async_copy(y_hbm.at[src8.at[0]], rows.at[0], gsem[0])
            gdescs = {0: gd}
            sdescs = {}
            for k_ in range(K):
                b, bn = k_ % 2, (k_ + 1) % 2
                if k_ < K - 1:
                    if k_ >= 1:
                        sdescs[k_ - 1].wait()
                    gdescs[k_ + 1] = pltpu.async_copy(
                        y_hbm.at[src8.at[k_ + 1]], rows.at[bn], gsem[bn])
                gdescs[k_].wait()
                sdescs[k_] = pltpu.async_copy(
                    rows.at[b], acc.at[dst8.at[k_]], ssem[b], add=True)
            return carry
        lax.fori_loop(0, n_groups, group, 0)

        for b in range(2):
            pltpu.make_async_copy(y_hbm.at[pl.ds(0, IDXW)],
                                  rows.at[b], ssem[b]).wait()

        plsc.subcore_barrier()
        pltpu.sync_copy(acc.at[pl.ds(off, zr)],
                        out_hbm.at[c, pl.ds(off, zr)])

    return k(y, src2, dst2)


def _mm2_tc(x, wl, wr):
    n = x.shape[0]
    h = wl.shape[0]

    def body(x_ref, wl_ref, wr_ref, y_ref, r_ref):
        xv = x_ref[...]
        y_ref[...] = jnp.concatenate(
            [_dot_t(xv, wl_ref[...]), jnp.ones((n, FW - h), jnp.float32)], axis=1)
        r_ref[...] = _dot_t(xv, wr_ref[...])

    return pl.pallas_call(
        body,
        out_shape=(jax.ShapeDtypeStruct((n, FW), jnp.float32),
                   jax.ShapeDtypeStruct((n, wr.shape[0]), jnp.float32)),
    )(x, wl, wr)


def _mean_from_acc(acc_ref, cnt_ref, n, h):
    csum = cnt_ref[0, 0:n, h:FW] + cnt_ref[1, 0:n, h:FW]
    cnt = jnp.sum(csum, axis=1, keepdims=True) * (1.0 / (FW - h))
    inv = 1.0 / jnp.maximum(cnt, 1.0)
    a = acc_ref[0, 0:n, 0:h] + acc_ref[1, 0:n, 0:h]
    return a * inv


def _comb_mm_tc(acc, cnt, b, r, wl, wr, n):
    hw = wl.shape[1]

    def body(acc_ref, cnt_ref, b_ref, r_ref, wl_ref, wr_ref, y_ref, r2_ref):
        m = _mean_from_acc(acc_ref, cnt_ref, n, hw)
        h = jnp.maximum(m + b_ref[...] + r_ref[...], 0.0)
        y_ref[...] = jnp.concatenate(
            [_dot_t(h, wl_ref[...]), jnp.zeros((n, FW - wl.shape[0]), jnp.float32)],
            axis=1)
        r2_ref[...] = _dot_t(h, wr_ref[...])

    return pl.pallas_call(
        body,
        out_shape=(jax.ShapeDtypeStruct((n, FW), jnp.float32),
                   jax.ShapeDtypeStruct((n, wr.shape[0]), jnp.float32)),
    )(acc, cnt, b, r, wl, wr)


def _comb_h_tc(acc, cnt, b, r, wr3, n):
    hw = r.shape[1]

    def body(acc_ref, cnt_ref, b_ref, r_ref, wr3_ref, h_ref, r3_ref):
        m = _mean_from_acc(acc_ref, cnt_ref, n, hw)
        h = jnp.maximum(m + b_ref[...] + r_ref[...], 0.0)
        h_ref[...] = jnp.concatenate(
            [h, jnp.zeros((n, FW - hw), jnp.float32)], axis=1)
        r3_ref[...] = _dot_t(h, wr3_ref[...])

    return pl.pallas_call(
        body,
        out_shape=(jax.ShapeDtypeStruct((n, FW), jnp.float32),
                   jax.ShapeDtypeStruct((n, wr3.shape[0]), jnp.float32)),
    )(acc, cnt, b, r, wr3)


def _final_tc(acc, cnt, b, r3, wl3, n):
    hw = wl3.shape[1]

    def body(acc_ref, cnt_ref, b_ref, r3_ref, wl3_ref, dx_ref):
        m = _mean_from_acc(acc_ref, cnt_ref, n, hw)
        dx_ref[...] = _dot_t(m, wl3_ref[...]) + b_ref[...] + r3_ref[...]

    return pl.pallas_call(
        body,
        out_shape=jax.ShapeDtypeStruct((n, wl3.shape[0]), jnp.float32),
    )(acc, cnt, b, r3, wl3)


def kernel(t, x, edge_index, Wl1, bl1, Wr1, Wl2, bl2, Wr2, Wl3, bl3, Wr3):
    del t
    n, d = x.shape
    e = edge_index.shape[1]

    ept = -(-e // (NW * IDXW * K)) * (IDXW * K)
    e_pad = ept * NW
    n_pad = -(-(n + 1) // (NS * 8)) * (NS * 8)

    ei = edge_index.astype(jnp.int32)
    perm = jnp.argsort(ei[0])
    srcs = ei[0][perm]
    dsts = ei[1][perm]
    pad = e_pad - e
    src2 = jnp.concatenate([srcs, jnp.zeros((pad,), jnp.int32)]).reshape(-1, IDXW)
    dst2 = jnp.concatenate([dsts, jnp.full((pad,), n, jnp.int32)]).reshape(-1, IDXW)

    bl1r = bl1.reshape(1, -1)
    bl2r = bl2.reshape(1, -1)
    bl3r = bl3.reshape(1, -1)

    y1, r1 = _mm2_tc(x, Wl1, Wr1)
    acc1 = _seg_sum_sc(y1, src2, dst2, n_pad, ept)
    y2, r2 = _comb_mm_tc(acc1, acc1, bl1r, r1, Wl2, Wr2, n)
    acc2 = _seg_sum_sc(y2, src2, dst2, n_pad, ept)
    h2, r3 = _comb_h_tc(acc2, acc1, bl2r, r2, Wr3, n)
    acc3 = _seg_sum_sc(h2, src2, dst2, n_pad, ept)
    dx = _final_tc(acc3, acc1, bl3r, r3, Wl3, n)
    return dx


# speedup vs baseline: 1.2851x; 1.2851x over previous
import functools

import jax
import jax.numpy as jnp
from jax import lax
from jax.experimental import pallas as pl
from jax.experimental.pallas import tpu as pltpu
from jax.experimental.pallas import tpu_sc as plsc

NC = 2
NS = 16
NW = NC * NS
LANES = 16
IDXW = 128
FW = 128


def _dot_t(a, w):
    return lax.dot_general(a, w, (((1,), (1,)), ((), ())),
                           preferred_element_type=jnp.float32)


K = 8


def _seg_sum_sc(y, src2, dst2, n_pad, ept):
    n_steps = ept // IDXW
    n_groups = n_steps // K
    zr = n_pad // NS
    mesh = plsc.VectorSubcoreMesh(core_axis_name="c", subcore_axis_name="s")

    @functools.partial(
        pl.kernel,
        out_type=jax.ShapeDtypeStruct((NC, n_pad, FW), jnp.float32),
        mesh=mesh,
        scratch_types=[
            pltpu.VMEM((K, IDXW), jnp.int32),
            pltpu.VMEM((K, IDXW), jnp.int32),
            pltpu.VMEM((2, IDXW, FW), jnp.float32),
            pltpu.VMEM_SHARED((n_pad, FW), jnp.float32),
            pltpu.SemaphoreType.DMA,
            pltpu.SemaphoreType.DMA,
            pltpu.SemaphoreType.DMA,
            pltpu.SemaphoreType.DMA,
        ],
    )
    def k(y_hbm, src_hbm, dst_hbm, out_hbm, src8, dst8, rows, acc,
          gsem0, gsem1, ssem0, ssem1):
        c = lax.axis_index("c")
        s = lax.axis_index("s")
        wid = s * NC + c
        off = pl.multiple_of(s * zr, 8)
        gsem = (gsem0, gsem1)
        ssem = (ssem0, ssem1)

        def zrow(i, carry):
            for j in range(FW // LANES):
                rows[0, i, pl.ds(j * LANES, LANES)] = jnp.zeros((LANES,),
                                                               jnp.float32)
            return carry
        lax.fori_loop(0, IDXW, zrow, 0)
        for zo in range(0, zr, IDXW):
            zn = min(IDXW, zr - zo)
            pltpu.sync_copy(rows.at[0, pl.ds(0, zn)],
                            acc.at[pl.ds(pl.multiple_of(off + zo, 8), zn)])
        plsc.subcore_barrier()

        tile_r0 = wid * (ept // IDXW)

        def group(g, carry):
            @pl.when(g > 0)
            def _():
                for b in range(2):
                    pltpu.make_async_copy(y_hbm.at[pl.ds(0, IDXW)],
                                          rows.at[b], ssem[b]).wait()

            r0 = pl.multiple_of(tile_r0 + g * K, 8)
            pltpu.sync_copy(src_hbm.at[pl.ds(r0, K)], src8)
            pltpu.sync_copy(dst_hbm.at[pl.ds(r0, K)], dst8)

            gd = pltpu.async_copy(y_hbm.at[src8.at[0]], rows.at[0], gsem[0])
            gdescs = {0: gd}
            sdescs = {}
            for k_ in range(K):
                b, bn = k_ % 2, (k_ + 1) % 2
                if k_ < K - 1:
                    if k_ >= 1:
                        sdescs[k_ - 1].wait()
                    gdescs[k_ + 1] = pltpu.async_copy(
                        y_hbm.at[src8.at[k_ + 1]], rows.at[bn], gsem[bn])
                gdescs[k_].wait()
                sdescs[k_] = pltpu.async_copy(
                    rows.at[b], acc.at[dst8.at[k_]], ssem[b], add=True)
            return carry
        lax.fori_loop(0, n_groups, group, 0)

        for b in range(2):
            pltpu.make_async_copy(y_hbm.at[pl.ds(0, IDXW)],
                                  rows.at[b], ssem[b]).wait()

        plsc.subcore_barrier()
        pltpu.sync_copy(acc.at[pl.ds(off, zr)],
                        out_hbm.at[c, pl.ds(off, zr)])

    return k(y, src2, dst2)


def _mm2_tc(x, wl, wr):
    n = x.shape[0]
    h = wl.shape[0]

    def body(x_ref, wl_ref, wr_ref, y_ref, r_ref):
        xv = x_ref[...]
        y_ref[...] = jnp.concatenate(
            [_dot_t(xv, wl_ref[...]), jnp.ones((n, FW - h), jnp.float32)], axis=1)
        r_ref[...] = _dot_t(xv, wr_ref[...])

    return pl.pallas_call(
        body,
        out_shape=(jax.ShapeDtypeStruct((n, FW), jnp.float32),
                   jax.ShapeDtypeStruct((n, wr.shape[0]), jnp.float32)),
    )(x, wl, wr)


def _mean_from_acc(acc_ref, cnt_ref, n, h):
    csum = cnt_ref[0, 0:n, h:FW] + cnt_ref[1, 0:n, h:FW]
    cnt = jnp.sum(csum, axis=1, keepdims=True) * (1.0 / (FW - h))
    inv = 1.0 / jnp.maximum(cnt, 1.0)
    a = acc_ref[0, 0:n, 0:h] + acc_ref[1, 0:n, 0:h]
    return a * inv


def _comb_mm_tc(acc, cnt, b, r, wl, wr, n):
    hw = wl.shape[1]

    def body(acc_ref, cnt_ref, b_ref, r_ref, wl_ref, wr_ref, y_ref, r2_ref):
        m = _mean_from_acc(acc_ref, cnt_ref, n, hw)
        h = jnp.maximum(m + b_ref[...] + r_ref[...], 0.0)
        y_ref[...] = jnp.concatenate(
            [_dot_t(h, wl_ref[...]), jnp.zeros((n, FW - wl.shape[0]), jnp.float32)],
            axis=1)
        r2_ref[...] = _dot_t(h, wr_ref[...])

    return pl.pallas_call(
        body,
        out_shape=(jax.ShapeDtypeStruct((n, FW), jnp.float32),
                   jax.ShapeDtypeStruct((n, wr.shape[0]), jnp.float32)),
    )(acc, cnt, b, r, wl, wr)


def _comb_h_tc(acc, cnt, b, r, wr3, n):
    hw = r.shape[1]

    def body(acc_ref, cnt_ref, b_ref, r_ref, wr3_ref, h_ref, r3_ref):
        m = _mean_from_acc(acc_ref, cnt_ref, n, hw)
        h = jnp.maximum(m + b_ref[...] + r_ref[...], 0.0)
        h_ref[...] = jnp.concatenate(
            [h, jnp.zeros((n, FW - hw), jnp.float32)], axis=1)
        r3_ref[...] = _dot_t(h, wr3_ref[...])

    return pl.pallas_call(
        body,
        out_shape=(jax.ShapeDtypeStruct((n, FW), jnp.float32),
                   jax.ShapeDtypeStruct((n, wr3.shape[0]), jnp.float32)),
    )(acc, cnt, b, r, wr3)


def _final_tc(acc, cnt, b, r3, wl3, n):
    hw = wl3.shape[1]

    def body(acc_ref, cnt_ref, b_ref, r3_ref, wl3_ref, dx_ref):
        m = _mean_from_acc(acc_ref, cnt_ref, n, hw)
        dx_ref[...] = _dot_t(m, wl3_ref[...]) + b_ref[...] + r3_ref[...]

    return pl.pallas_call(
        body,
        out_shape=jax.ShapeDtypeStruct((n, wl3.shape[0]), jnp.float32),
    )(acc, cnt, b, r3, wl3)


def kernel(t, x, edge_index, Wl1, bl1, Wr1, Wl2, bl2, Wr2, Wl3, bl3, Wr3):
    del t
    n, d = x.shape
    e = edge_index.shape[1]

    ept = -(-e // (NW * IDXW * K)) * (IDXW * K)
    e_pad = ept * NW
    n_pad = -(-(n + 1) // (NS * 8)) * (NS * 8)

    ei = edge_index.astype(jnp.int32)
    pad = e_pad - e
    src2 = jnp.concatenate([ei[0], jnp.zeros((pad,), jnp.int32)]).reshape(-1, IDXW)
    dst2 = jnp.concatenate([ei[1], jnp.full((pad,), n, jnp.int32)]).reshape(-1, IDXW)

    bl1r = bl1.reshape(1, -1)
    bl2r = bl2.reshape(1, -1)
    bl3r = bl3.reshape(1, -1)

    y1, r1 = _mm2_tc(x, Wl1, Wr1)
    acc1 = _seg_sum_sc(y1, src2, dst2, n_pad, ept)
    y2, r2 = _comb_mm_tc(acc1, acc1, bl1r, r1, Wl2, Wr2, n)
    acc2 = _seg_sum_sc(y2, src2, dst2, n_pad, ept)
    h2, r3 = _comb_h_tc(acc2, acc1, bl2r, r2, Wr3, n)
    acc3 = _seg_sum_sc(h2, src2, dst2, n_pad, ept)
    dx = _final_tc(acc3, acc1, bl3r, r3, Wl3, n)
    return dx
